# baseline (speedup 1.0000x reference)
import jax
import jax.numpy as jnp
from jax.experimental import pallas as pl
from jax.experimental.pallas import tpu as pltpu

S = 4096
E = 8
C = 640
T = 1024


def _router_kernel(x_ref, comb_ref, cr_ref, base_s):
    i = pl.program_id(0)

    @pl.when(i == 0)
    def _init():
        base_s[:, :] = jnp.zeros((E, 1), jnp.int32)

    x = x_ref[:, :]
    mx = jnp.max(x, axis=0, keepdims=True)
    denom = jnp.sum(jnp.exp(x - mx), axis=0, keepdims=True)
    w = 1.0 / denom
    eidx = jax.lax.broadcasted_iota(jnp.int32, (E, T), 0)
    top1 = jnp.min(jnp.where(x == mx, eidx, E), axis=0, keepdims=True)
    sel = eidx == top1
    onehot = sel.astype(jnp.int32)

    c = onehot
    sh = 1
    while sh < T:
        c = c + jnp.concatenate(
            [jnp.zeros((E, sh), jnp.int32), c[:, : T - sh]], axis=1)
        sh *= 2
    ranks = base_s[:, :] + c - 1
    base_s[:, :] = base_s[:, :] + c[:, T - 1 : T]

    cr = jnp.where(sel & (ranks < C), ranks, -1)
    crt = cr.T
    cr_ref[:, :] = crt

    crb = crt.reshape(T, E, 1)
    wb = jnp.broadcast_to(w, (E, T)).T.reshape(T, E, 1)
    c_iota = jax.lax.broadcasted_iota(jnp.int32, (T, E, C), 2)
    comb_ref[:, :, :] = jnp.where(c_iota == crb, wb, 0.0)


def kernel(inputs):
    comb, cr = pl.pallas_call(
        _router_kernel,
        grid=(S // T,),
        in_specs=[pl.BlockSpec((E, T), lambda i: (0, i))],
        out_specs=[
            pl.BlockSpec((T, E, C), lambda i: (i, 0, 0)),
            pl.BlockSpec((T, E), lambda i: (i, 0)),
        ],
        out_shape=[
            jax.ShapeDtypeStruct((S, E, C), jnp.float32),
            jax.ShapeDtypeStruct((S, E), jnp.int32),
        ],
        scratch_shapes=[
            pltpu.VMEM((E, 1), jnp.int32),
        ],
    )(inputs.T)
    sec_mask = cr[:, :, None] == jnp.arange(C, dtype=jnp.int32)[None, None, :]
    return comb, sec_mask


# speedup vs baseline: 1.0456x; 1.0456x over previous
import jax
import jax.numpy as jnp
from jax.experimental import pallas as pl
from jax.experimental.pallas import tpu as pltpu

S = 4096
E = 8
C = 640
T = 512


def _router_kernel(x_ref, comb_ref, cr_ref, base_s):
    i = pl.program_id(0)

    @pl.when(i == 0)
    def _init():
        base_s[:, :] = jnp.zeros((E, 1), jnp.int32)

    x = x_ref[:, :]
    mx = jnp.max(x, axis=0, keepdims=True)
    denom = jnp.sum(jnp.exp(x - mx), axis=0, keepdims=True)
    w = 1.0 / denom
    eidx = jax.lax.broadcasted_iota(jnp.int32, (E, T), 0)
    top1 = jnp.min(jnp.where(x == mx, eidx, E), axis=0, keepdims=True)
    sel = eidx == top1
    onehot = sel.astype(jnp.int32)

    c = onehot
    sh = 1
    while sh < T:
        c = c + jnp.concatenate(
            [jnp.zeros((E, sh), jnp.int32), c[:, : T - sh]], axis=1)
        sh *= 2
    ranks = base_s[:, :] + c - 1
    base_s[:, :] = base_s[:, :] + c[:, T - 1 : T]

    cr = jnp.where(sel & (ranks < C), ranks, -1)
    crt = cr.T
    cr_ref[:, :] = crt

    crb = crt.reshape(T, E, 1)
    wb = jnp.broadcast_to(w, (E, T)).T.reshape(T, E, 1)
    c_iota = jax.lax.broadcasted_iota(jnp.int32, (T, E, C), 2)
    comb_ref[:, :, :] = jnp.where(c_iota == crb, wb, 0.0)


def kernel(inputs):
    comb, cr = pl.pallas_call(
        _router_kernel,
        grid=(S // T,),
        in_specs=[pl.BlockSpec((E, T), lambda i: (0, i))],
        out_specs=[
            pl.BlockSpec((T, E, C), lambda i: (i, 0, 0)),
            pl.BlockSpec((T, E), lambda i: (i, 0)),
        ],
        out_shape=[
            jax.ShapeDtypeStruct((S, E, C), jnp.float32),
            jax.ShapeDtypeStruct((S, E), jnp.int32),
        ],
        scratch_shapes=[
            pltpu.VMEM((E, 1), jnp.int32),
        ],
    )(inputs.T)
    sec_mask = cr[:, :, None] == jnp.arange(C, dtype=jnp.int32)[None, None, :]
    return comb, sec_mask
